# Initial kernel scaffold; baseline (speedup 1.0000x reference)
import jax
import jax.numpy as jnp
from jax.experimental import pallas as pl


def kernel(cosine, label):
    raise NotImplementedError("write your pallas kernel here")


# speedup vs baseline: 2.0599x; 2.0599x over previous
import functools

import jax
import jax.numpy as jnp
from jax.experimental import pallas as pl
from jax.experimental.pallas import tpu as pltpu

B = 1024
V = 100000
MARGIN = 0.3
SCALE = 32.0

BLOCK_V = 4096
NUM_BLOCKS = (V + BLOCK_V - 1) // BLOCK_V


def _amloss_kernel(cosine_ref, label_ref, out_ref, m_ref, s_ref, picked_ref):
    i = pl.program_id(0)

    @pl.when(i == 0)
    def _init():
        m_ref[:, :] = jnp.full((B, 1), -jnp.inf, jnp.float32)
        s_ref[:, :] = jnp.zeros((B, 1), jnp.float32)
        picked_ref[:, :] = jnp.zeros((B, 1), jnp.float32)

    x = cosine_ref[:, :] * SCALE
    col0 = i * BLOCK_V
    cols = jax.lax.broadcasted_iota(jnp.int32, (B, BLOCK_V), 1) + col0
    is_label = cols == label_ref[:, :]
    valid = cols < V
    xm = jnp.where(is_label, x - (SCALE * MARGIN), x)
    xm = jnp.where(valid, xm, -jnp.inf)

    bm = jnp.max(xm, axis=1, keepdims=True)
    m_prev = m_ref[:, :]
    m_new = jnp.maximum(m_prev, bm)
    alpha = jnp.exp(m_prev - m_new)
    s_ref[:, :] = s_ref[:, :] * alpha + jnp.sum(
        jnp.exp(xm - m_new), axis=1, keepdims=True
    )
    m_ref[:, :] = m_new
    picked_ref[:, :] += jnp.sum(
        jnp.where(is_label, xm, 0.0), axis=1, keepdims=True
    )

    @pl.when(i == NUM_BLOCKS - 1)
    def _finish():
        lse = m_ref[:, :] + jnp.log(s_ref[:, :])
        out_ref[:, :] = jnp.sum(
            lse - picked_ref[:, :], axis=0, keepdims=True
        ) * (1.0 / B)


@functools.partial(jax.jit, static_argnames=("interpret",))
def _amloss(cosine, label, interpret=False):
    label2d = label.reshape(B, 1).astype(jnp.int32)
    out = pl.pallas_call(
        _amloss_kernel,
        grid=(NUM_BLOCKS,),
        in_specs=[
            pl.BlockSpec((B, BLOCK_V), lambda i: (0, i)),
            pl.BlockSpec((B, 1), lambda i: (0, 0)),
        ],
        out_specs=pl.BlockSpec((1, 1), lambda i: (0, 0)),
        out_shape=jax.ShapeDtypeStruct((1, 1), jnp.float32),
        scratch_shapes=[
            pltpu.VMEM((B, 1), jnp.float32),
            pltpu.VMEM((B, 1), jnp.float32),
            pltpu.VMEM((B, 1), jnp.float32),
        ],
        interpret=interpret,
    )(cosine, label2d)
    return out[0, 0]


def kernel(cosine, label):
    return _amloss(cosine, label)
